# baseline (speedup 1.0000x reference)
import jax
import jax.numpy as jnp
from jax.experimental import pallas as pl
from jax.experimental.pallas import tpu as pltpu

N, D, H = 10000, 128, 256
BC = 100
SUB = 4
BM = BC * SUB
NB = 8
NCHUNKS = N // BC


def _chunk_copy(adj_hbm, buf, sem, g, slot):
    return pltpu.make_async_copy(
        adj_hbm.at[g], buf.at[slot], sem.at[slot])


def _gcn_kernel(adj_hbm, x_ref, W1_ref, W2_ref, b_ref, out_ref, buf, sem):
    m = pl.program_id(0)

    @pl.when(m == 0)
    def _():
        for j in range(NB - 1):
            _chunk_copy(adj_hbm, buf, sem, j, j).start()

    for s in range(SUB):
        g = m * SUB + s
        nxt = g + NB - 1

        @pl.when(nxt < NCHUNKS)
        def _():
            _chunk_copy(adj_hbm, buf, sem, nxt, nxt % NB).start()

        _chunk_copy(adj_hbm, buf, sem, g, g % NB).wait()
        agg = jnp.dot(buf[g % NB], x_ref[:], preferred_element_type=jnp.float32)
        x_self = x_ref[pl.ds(g * BC, BC), :]
        z = jnp.dot(x_self, W1_ref[:], preferred_element_type=jnp.float32)
        z += jnp.dot(agg, W2_ref[:], preferred_element_type=jnp.float32)
        z += b_ref[:]
        out_ref[pl.ds(s * BC, BC), :] = jnp.maximum(z, 0.0)


def kernel(x, adj, W, b):
    W1 = W[:D]
    W2 = W[D:]
    b2 = b.reshape(1, H)
    grid = (N // BM,)
    return pl.pallas_call(
        _gcn_kernel,
        grid=grid,
        in_specs=[
            pl.BlockSpec(memory_space=pl.ANY),
            pl.BlockSpec((N, D), lambda m: (0, 0)),
            pl.BlockSpec((D, H), lambda m: (0, 0)),
            pl.BlockSpec((D, H), lambda m: (0, 0)),
            pl.BlockSpec((1, H), lambda m: (0, 0)),
        ],
        out_specs=pl.BlockSpec((BM, H), lambda m: (m, 0)),
        out_shape=jax.ShapeDtypeStruct((N, H), jnp.float32),
        scratch_shapes=[
            pltpu.VMEM((NB, BC, N), jnp.float32),
            pltpu.SemaphoreType.DMA((NB,)),
        ],
        compiler_params=pltpu.CompilerParams(
            dimension_semantics=("arbitrary",),
        ),
    )(adj.reshape(NCHUNKS, BC, N), x, W1, W2, b2)


# speedup vs baseline: 1.0045x; 1.0045x over previous
import jax
import jax.numpy as jnp
from jax.experimental import pallas as pl
from jax.experimental.pallas import tpu as pltpu

N, D, H = 10000, 128, 256
BC = 100
SUB = 4
BM = BC * SUB
NCHUNKS = N // BC


def _chunk_copy(adj_hbm, buf, sem, g, slot):
    return pltpu.make_async_copy(
        adj_hbm.at[g], buf.at[slot], sem.at[slot])


def _gcn_kernel(adj_hbm, x_ref, W1_ref, W2_ref, b_ref, out_ref, buf, sem):
    m = pl.program_id(0)

    @pl.when(m == 0)
    def _():
        for j in range(SUB - 1):
            _chunk_copy(adj_hbm, buf, sem, j, j).start()

    for s in range(SUB):
        g = m * SUB + s
        nxt = g + SUB - 1
        nxt_slot = (s + SUB - 1) % SUB

        @pl.when(nxt < NCHUNKS)
        def _():
            _chunk_copy(adj_hbm, buf, sem, nxt, nxt_slot).start()

        _chunk_copy(adj_hbm, buf, sem, g, s).wait()
        agg = jnp.dot(buf[s], x_ref[:], preferred_element_type=jnp.float32)
        x_self = x_ref[pl.ds(g * BC, BC), :]
        z = jnp.dot(x_self, W1_ref[:], preferred_element_type=jnp.float32)
        z += jnp.dot(agg, W2_ref[:], preferred_element_type=jnp.float32)
        z += b_ref[:]
        out_ref[pl.ds(s * BC, BC), :] = jnp.maximum(z, 0.0)


def kernel(x, adj, W, b):
    W1 = W[:D]
    W2 = W[D:]
    b2 = b.reshape(1, H)
    grid = (N // BM,)
    return pl.pallas_call(
        _gcn_kernel,
        grid=grid,
        in_specs=[
            pl.BlockSpec(memory_space=pl.ANY),
            pl.BlockSpec((N, D), lambda m: (0, 0)),
            pl.BlockSpec((D, H), lambda m: (0, 0)),
            pl.BlockSpec((D, H), lambda m: (0, 0)),
            pl.BlockSpec((1, H), lambda m: (0, 0)),
        ],
        out_specs=pl.BlockSpec((BM, H), lambda m: (m, 0)),
        out_shape=jax.ShapeDtypeStruct((N, H), jnp.float32),
        scratch_shapes=[
            pltpu.VMEM((SUB, BC, N), jnp.float32),
            pltpu.SemaphoreType.DMA((SUB,)),
        ],
        compiler_params=pltpu.CompilerParams(
            dimension_semantics=("arbitrary",),
        ),
    )(adj.reshape(NCHUNKS, BC, N), x, W1, W2, b2)


# speedup vs baseline: 3.8884x; 3.8709x over previous
import jax
import jax.numpy as jnp
from jax.experimental import pallas as pl
from jax.experimental.pallas import tpu as pltpu

N, D, H = 10000, 128, 256
BM = 400


def _gcn_kernel(adj_ref, x_ref, W1_ref, W2_ref, b_ref, out_ref):
    m = pl.program_id(0)
    x_self = x_ref[pl.ds(m * BM, BM), :]
    agg = jnp.dot(adj_ref[:], x_ref[:], preferred_element_type=jnp.float32)
    z = jnp.dot(x_self, W1_ref[:], preferred_element_type=jnp.float32)
    z += jnp.dot(agg, W2_ref[:], preferred_element_type=jnp.float32)
    z += b_ref[:]
    out_ref[:] = jnp.maximum(z, 0.0)


def kernel(x, adj, W, b):
    W1 = W[:D]
    W2 = W[D:]
    b2 = b.reshape(1, H)
    grid = (N // BM,)
    return pl.pallas_call(
        _gcn_kernel,
        grid=grid,
        in_specs=[
            pl.BlockSpec((BM, N), lambda m: (m, 0)),
            pl.BlockSpec((N, D), lambda m: (0, 0)),
            pl.BlockSpec((D, H), lambda m: (0, 0)),
            pl.BlockSpec((D, H), lambda m: (0, 0)),
            pl.BlockSpec((1, H), lambda m: (0, 0)),
        ],
        out_specs=pl.BlockSpec((BM, H), lambda m: (m, 0)),
        out_shape=jax.ShapeDtypeStruct((N, H), jnp.float32),
        compiler_params=pltpu.CompilerParams(
            dimension_semantics=("parallel",),
        ),
    )(adj, x, W1, W2, b2)
